# baseline (speedup 1.0000x reference)
import functools

import jax
import jax.numpy as jnp
from jax.experimental import pallas as pl
from jax.experimental.pallas import tpu as pltpu

NUM_EMBEDDINGS = 1024
EMBEDDING_DIM = 64
COMMITMENT_COST = 0.25
CONTRIB_RATE = 0.05

ROWS = 9216
BLK = 3072
GRID = ROWS // BLK
_LOSS_SCALE = (1.0 + COMMITMENT_COST) / float(ROWS * EMBEDDING_DIM)


def _vq_kernel(x_ref, w_ref, out_ref, idx_ref, loss_ref,
               wsq_ref, w2_ref, ids_ref):
    i = pl.program_id(0)
    x = x_ref[...]
    w = w_ref[...]

    @pl.when(i == 0)
    def _():
        wsq_ref[...] = jnp.sum(w * w, axis=1)[None, :]
        w2_ref[...] = w * -2.0
        ids_ref[...] = jax.lax.broadcasted_iota(
            jnp.int32, (1, NUM_EMBEDDINGS), 1).astype(jnp.float32)

    HB = BLK // 32
    parts = []
    for h in range(32):
        rs = pl.ds(h * HB, HB)
        xh = x[h * HB:(h + 1) * HB, :]

        xsq = jnp.sum(xh * xh, axis=1, keepdims=True)
        xw2 = jax.lax.dot_general(
            xh, w2_ref[...], (((1,), (1,)), ((), ())),
            preferred_element_type=jnp.float32)
        dist = (xsq + wsq_ref[...]) + xw2

        dmin = jnp.min(dist, axis=1, keepdims=True)
        m = dist == dmin
        idxf = jnp.min(jnp.where(m, ids_ref[...], float(NUM_EMBEDDINGS)),
                       axis=1, keepdims=True)
        idx_ref[rs, :] = idxf.astype(jnp.int32)

        enc = jnp.where(m, 1.0, 0.0)
        quant = jax.lax.dot_general(
            enc, w, (((1,), (0,)), ((), ())),
            preferred_element_type=jnp.float32)
        out_ref[rs, :] = (xh * (1.0 - CONTRIB_RATE)
                          + (quant - xh) * CONTRIB_RATE)

        parts.append(jnp.sum(dmin, axis=(0, 1), keepdims=True))

    while len(parts) > 1:
        parts = [parts[k] + parts[k + 1] for k in range(0, len(parts), 2)]
    part = parts[0]
    prev = jnp.where(i == 0, 0.0, loss_ref[...])
    acc = prev + part
    loss_ref[...] = jnp.where(i == GRID - 1, acc * _LOSS_SCALE, acc)


@functools.partial(jax.jit, static_argnames=())
def kernel(inputs, W):
    input_shape = inputs.shape
    flat = inputs.reshape(ROWS, EMBEDDING_DIM)
    out, idx, loss = pl.pallas_call(
        _vq_kernel,
        grid=(GRID,),
        in_specs=[
            pl.BlockSpec((BLK, EMBEDDING_DIM), lambda i: (i, 0)),
            pl.BlockSpec((NUM_EMBEDDINGS, EMBEDDING_DIM), lambda i: (0, 0)),
        ],
        out_specs=[
            pl.BlockSpec((BLK, EMBEDDING_DIM), lambda i: (i, 0)),
            pl.BlockSpec((BLK, 1), lambda i: (i, 0)),
            pl.BlockSpec((1, 1), lambda i: (0, 0)),
        ],
        out_shape=[
            jax.ShapeDtypeStruct((ROWS, EMBEDDING_DIM), jnp.float32),
            jax.ShapeDtypeStruct((ROWS, 1), jnp.int32),
            jax.ShapeDtypeStruct((1, 1), jnp.float32),
        ],
        scratch_shapes=[
            pltpu.VMEM((1, NUM_EMBEDDINGS), jnp.float32),
            pltpu.VMEM((NUM_EMBEDDINGS, EMBEDDING_DIM), jnp.float32),
            pltpu.VMEM((1, NUM_EMBEDDINGS), jnp.float32),
        ],
        compiler_params=pltpu.CompilerParams(
            dimension_semantics=("arbitrary",)),
    )(flat, W)
    return out.reshape(input_shape), idx, loss[0, 0]


# speedup vs baseline: 1.4935x; 1.4935x over previous
import functools

import jax
import jax.numpy as jnp
from jax.experimental import pallas as pl
from jax.experimental.pallas import tpu as pltpu

NUM_EMBEDDINGS = 1024
EMBEDDING_DIM = 64
COMMITMENT_COST = 0.25
CONTRIB_RATE = 0.05

ROWS = 9216
BLK = 4608
GRID = ROWS // BLK
_LOSS_SCALE = (1.0 + COMMITMENT_COST) / float(ROWS * EMBEDDING_DIM)


def _vq_kernel(x_ref, w_ref, out_ref, idx_ref, loss_ref,
               wsq_ref, w2_ref, ids_ref):
    i = pl.program_id(0)
    x = x_ref[...]
    w = w_ref[...]

    @pl.when(i == 0)
    def _():
        wsq_ref[...] = jnp.sum(w * w, axis=1)[None, :]
        w2_ref[...] = w * -2.0
        ids_ref[...] = jax.lax.broadcasted_iota(
            jnp.int32, (1, NUM_EMBEDDINGS), 1).astype(jnp.float32)

    HB = BLK // 24
    parts = []
    for h in range(24):
        rs = pl.ds(h * HB, HB)
        xh = x[h * HB:(h + 1) * HB, :]

        xsq = jnp.sum(xh * xh, axis=1, keepdims=True)
        xw2 = jax.lax.dot_general(
            xh, w2_ref[...], (((1,), (1,)), ((), ())),
            preferred_element_type=jnp.float32)
        dist = (xsq + wsq_ref[...]) + xw2

        dmin = jnp.min(dist, axis=1, keepdims=True)
        m = dist == dmin
        idxf = jnp.min(jnp.where(m, ids_ref[...], float(NUM_EMBEDDINGS)),
                       axis=1, keepdims=True)
        idx_ref[rs, :] = idxf.astype(jnp.int32)

        enc = jnp.where(m, 1.0, 0.0)
        quant = jax.lax.dot_general(
            enc, w, (((1,), (0,)), ((), ())),
            preferred_element_type=jnp.float32)
        out_ref[rs, :] = (xh * (1.0 - CONTRIB_RATE)
                          + (quant - xh) * CONTRIB_RATE)

        parts.append(jnp.sum(dmin, axis=(0, 1), keepdims=True))

    while len(parts) > 1:
        nxt = [parts[k] + parts[k + 1] for k in range(0, len(parts) - 1, 2)]
        if len(parts) % 2:
            nxt.append(parts[-1])
        parts = nxt
    part = parts[0]
    prev = jnp.where(i == 0, 0.0, loss_ref[...])
    acc = prev + part
    loss_ref[...] = jnp.where(i == GRID - 1, acc * _LOSS_SCALE, acc)


@functools.partial(jax.jit, static_argnames=())
def kernel(inputs, W):
    input_shape = inputs.shape
    flat = inputs.reshape(ROWS, EMBEDDING_DIM)
    out, idx, loss = pl.pallas_call(
        _vq_kernel,
        grid=(GRID,),
        in_specs=[
            pl.BlockSpec((BLK, EMBEDDING_DIM), lambda i: (i, 0)),
            pl.BlockSpec((NUM_EMBEDDINGS, EMBEDDING_DIM), lambda i: (0, 0)),
        ],
        out_specs=[
            pl.BlockSpec((BLK, EMBEDDING_DIM), lambda i: (i, 0)),
            pl.BlockSpec((BLK, 1), lambda i: (i, 0)),
            pl.BlockSpec((1, 1), lambda i: (0, 0)),
        ],
        out_shape=[
            jax.ShapeDtypeStruct((ROWS, EMBEDDING_DIM), jnp.float32),
            jax.ShapeDtypeStruct((ROWS, 1), jnp.int32),
            jax.ShapeDtypeStruct((1, 1), jnp.float32),
        ],
        scratch_shapes=[
            pltpu.VMEM((1, NUM_EMBEDDINGS), jnp.float32),
            pltpu.VMEM((NUM_EMBEDDINGS, EMBEDDING_DIM), jnp.float32),
            pltpu.VMEM((1, NUM_EMBEDDINGS), jnp.float32),
        ],
        compiler_params=pltpu.CompilerParams(
            dimension_semantics=("arbitrary",)),
    )(flat, W)
    return out.reshape(input_shape), idx, loss[0, 0]


# speedup vs baseline: 1.5090x; 1.0104x over previous
import functools

import jax
import jax.numpy as jnp
from jax.experimental import pallas as pl
from jax.experimental.pallas import tpu as pltpu

NUM_EMBEDDINGS = 1024
EMBEDDING_DIM = 64
COMMITMENT_COST = 0.25
CONTRIB_RATE = 0.05

ROWS = 9216
BLK = 3072
GRID = ROWS // BLK
_LOSS_SCALE = (1.0 + COMMITMENT_COST) / float(ROWS * EMBEDDING_DIM)


def _vq_kernel(x_ref, w_ref, out_ref, idx_ref, loss_ref,
               wsq_ref, w2_ref, ids_ref):
    i = pl.program_id(0)
    x = x_ref[...]
    w = w_ref[...]

    @pl.when(i == 0)
    def _():
        wsq_ref[...] = jnp.sum(w * w, axis=1)[None, :]
        w2_ref[...] = w * -2.0
        ids_ref[...] = jax.lax.broadcasted_iota(
            jnp.int32, (1, NUM_EMBEDDINGS), 1).astype(jnp.float32)

    HB = BLK // 16
    parts = []
    for h in range(16):
        rs = pl.ds(h * HB, HB)
        xh = x[h * HB:(h + 1) * HB, :]

        xsq = jnp.sum(xh * xh, axis=1, keepdims=True)
        xw2 = jax.lax.dot_general(
            xh, w2_ref[...], (((1,), (1,)), ((), ())),
            preferred_element_type=jnp.float32)
        dist = (xsq + wsq_ref[...]) + xw2

        dmin = jnp.min(dist, axis=1, keepdims=True)
        m = dist == dmin
        idxf = jnp.min(jnp.where(m, ids_ref[...], float(NUM_EMBEDDINGS)),
                       axis=1, keepdims=True)
        idx_ref[rs, :] = idxf.astype(jnp.int32)

        enc = jnp.where(m, 1.0, 0.0)
        quant = jax.lax.dot_general(
            enc, w, (((1,), (0,)), ((), ())),
            preferred_element_type=jnp.float32)
        out_ref[rs, :] = (xh * (1.0 - CONTRIB_RATE)
                          + (quant - xh) * CONTRIB_RATE)

        parts.append(jnp.sum(dmin, axis=(0, 1), keepdims=True))

    while len(parts) > 1:
        nxt = [parts[k] + parts[k + 1] for k in range(0, len(parts) - 1, 2)]
        if len(parts) % 2:
            nxt.append(parts[-1])
        parts = nxt
    part = parts[0]
    prev = jnp.where(i == 0, 0.0, loss_ref[...])
    acc = prev + part
    loss_ref[...] = jnp.where(i == GRID - 1, acc * _LOSS_SCALE, acc)


@functools.partial(jax.jit, static_argnames=())
def kernel(inputs, W):
    input_shape = inputs.shape
    flat = inputs.reshape(ROWS, EMBEDDING_DIM)
    out, idx, loss = pl.pallas_call(
        _vq_kernel,
        grid=(GRID,),
        in_specs=[
            pl.BlockSpec((BLK, EMBEDDING_DIM), lambda i: (i, 0)),
            pl.BlockSpec((NUM_EMBEDDINGS, EMBEDDING_DIM), lambda i: (0, 0)),
        ],
        out_specs=[
            pl.BlockSpec((BLK, EMBEDDING_DIM), lambda i: (i, 0)),
            pl.BlockSpec((BLK, 1), lambda i: (i, 0)),
            pl.BlockSpec((1, 1), lambda i: (0, 0)),
        ],
        out_shape=[
            jax.ShapeDtypeStruct((ROWS, EMBEDDING_DIM), jnp.float32),
            jax.ShapeDtypeStruct((ROWS, 1), jnp.int32),
            jax.ShapeDtypeStruct((1, 1), jnp.float32),
        ],
        scratch_shapes=[
            pltpu.VMEM((1, NUM_EMBEDDINGS), jnp.float32),
            pltpu.VMEM((NUM_EMBEDDINGS, EMBEDDING_DIM), jnp.float32),
            pltpu.VMEM((1, NUM_EMBEDDINGS), jnp.float32),
        ],
        compiler_params=pltpu.CompilerParams(
            dimension_semantics=("arbitrary",)),
    )(flat, W)
    return out.reshape(input_shape), idx, loss[0, 0]
